# baseline (speedup 1.0000x reference)
import jax
import jax.numpy as jnp
from jax import lax
from jax.experimental import pallas as pl
from jax.experimental.pallas import tpu as pltpu
from jax.experimental.pallas import tpu_sc as plsc

V = 256
D_G = 512
T = 2048
P = 16
K = 128
B = 4

_info = plsc.get_sparse_core_info()
NS, L = _info.num_subcores, _info.num_lanes
C = T // NS
R = 32
SB = 2
NSUB = (SB * C) // R
NRING = 3


def _sc_body(bytes_hbm, emb_hbm, pos_hbm, out_hbm,
             idx_buf, pbuf, ring0, ring1, ring2,
             psem, gsem0, gsem1, gsem2, osem0, osem1, osem2):
    s = lax.axis_index("s")
    t0 = s * C

    rings = (ring0, ring1, ring2)
    gsems = (gsem0, gsem1, gsem2)
    osems = (osem0, osem1, osem2)

    pos_dma = pltpu.async_copy(pos_hbm.at[pl.ds(t0, C)], pbuf, psem)
    for b in range(SB):
        pltpu.sync_copy(bytes_hbm.at[b, pl.ds(t0, C)], idx_buf.at[b])

    def issue_gather(sn):
        b, h = divmod(sn, C // R)
        slot = sn % NRING
        idx = idx_buf.at[b, pl.ds(h * R, R)]
        return pltpu.async_copy(emb_hbm.at[idx], rings[slot], gsems[slot])

    gather_dmas = [None] * NSUB
    out_dmas = [None] * NSUB
    for sn in range(2):
        gather_dmas[sn] = issue_gather(sn)
    pos_dma.wait()

    for sn in range(NSUB):
        b, h = divmod(sn, C // R)
        slot = sn % NRING
        buf = rings[slot]
        gather_dmas[sn].wait()

        @plsc.parallel_loop(0, R)
        def add_row(r, buf=buf, h=h):
            for j in range(D_G // L):
                sl = pl.ds(j * L, L)
                plsc.addupdate(buf.at[r, sl], pbuf[h * R + r, sl])

        out_dmas[sn] = pltpu.async_copy(
            buf, out_hbm.at[pl.ds(b * T + t0 + h * R, R)], osems[slot])

        nxt = sn + 2
        if nxt < NSUB:
            if nxt >= NRING:
                out_dmas[nxt - NRING].wait()
            gather_dmas[nxt] = issue_gather(nxt)

    for sn in range(NSUB - min(NRING, NSUB), NSUB):
        if out_dmas[sn] is not None:
            out_dmas[sn].wait()


def _make_sc():
    return pl.kernel(
        _sc_body,
        out_type=jax.ShapeDtypeStruct((SB * T, D_G), jnp.float32),
        mesh=plsc.VectorSubcoreMesh(core_axis_name="c", subcore_axis_name="s",
                                    num_cores=1),
        scratch_types=(
            [pltpu.VMEM((SB, C), jnp.int32),
             pltpu.VMEM((C, D_G), jnp.float32)]
            + [pltpu.VMEM((R, D_G), jnp.float32) for _ in range(NRING)]
            + [pltpu.SemaphoreType.DMA for _ in range(1 + 2 * NRING)]
        ),
    )


@jax.jit
def _patch_embed(bytes_, emb, pos):
    a = _make_sc()(bytes_[:SB], emb, pos)
    b = _make_sc()(bytes_[SB:], emb, pos)
    out = jnp.concatenate([a.reshape(SB, K, P * D_G),
                           b.reshape(SB, K, P * D_G)], axis=0)
    return out


def kernel(bytes, emb, pos):
    return _patch_embed(bytes, emb, pos)


# speedup vs baseline: 1.1544x; 1.1544x over previous
import jax
import jax.numpy as jnp
from jax import lax
from jax.experimental import pallas as pl
from jax.experimental.pallas import tpu as pltpu
from jax.experimental.pallas import tpu_sc as plsc

V = 256
D_G = 512
T = 2048
P = 16
K = 128
B = 4

_info = plsc.get_sparse_core_info()
NS, L = _info.num_subcores, _info.num_lanes
C = T // NS
R = 32
NSUB = (B * C) // R
NRING = 3


def _sc_body(bytes_hbm, emb_hbm, pos_hbm, out_hbm,
             idx_buf, pbuf, ring0, ring1, ring2,
             psem, gsem0, gsem1, gsem2, osem0, osem1, osem2):
    s = lax.axis_index("s")
    t0 = s * C

    rings = (ring0, ring1, ring2)
    gsems = (gsem0, gsem1, gsem2)
    osems = (osem0, osem1, osem2)

    pos_dma = pltpu.async_copy(pos_hbm.at[pl.ds(t0, C)], pbuf, psem)
    for b in range(B):
        pltpu.sync_copy(bytes_hbm.at[b, pl.ds(t0, C)], idx_buf.at[b])

    def issue_gather(sn):
        b, h = divmod(sn, C // R)
        slot = sn % NRING
        idx = idx_buf.at[b, pl.ds(h * R, R)]
        return pltpu.async_copy(emb_hbm.at[idx], rings[slot], gsems[slot])

    gather_dmas = [None] * NSUB
    out_dmas = [None] * NSUB
    for sn in range(2):
        gather_dmas[sn] = issue_gather(sn)
    pos_dma.wait()

    for sn in range(NSUB):
        b, h = divmod(sn, C // R)
        slot = sn % NRING
        buf = rings[slot]
        gather_dmas[sn].wait()

        @plsc.parallel_loop(0, R)
        def add_row(r, buf=buf, h=h):
            for j in range(D_G // L):
                sl = pl.ds(j * L, L)
                plsc.addupdate(buf.at[r, sl], pbuf[h * R + r, sl])

        out_dmas[sn] = pltpu.async_copy(
            buf, out_hbm.at[pl.ds(b * T + t0 + h * R, R)], osems[slot])

        nxt = sn + 2
        if nxt < NSUB:
            if nxt >= NRING:
                out_dmas[nxt - NRING].wait()
            gather_dmas[nxt] = issue_gather(nxt)

    for sn in range(NSUB - NRING, NSUB):
        if out_dmas[sn] is not None:
            out_dmas[sn].wait()


@jax.jit
def _patch_embed(bytes_, emb, pos):
    run = pl.kernel(
        _sc_body,
        out_type=jax.ShapeDtypeStruct((B * T, D_G), jnp.float32),
        mesh=plsc.VectorSubcoreMesh(core_axis_name="c", subcore_axis_name="s",
                                    num_cores=1),
        scratch_types=(
            [pltpu.VMEM((B, C), jnp.int32),
             pltpu.VMEM((C, D_G), jnp.float32)]
            + [pltpu.VMEM((R, D_G), jnp.float32) for _ in range(NRING)]
            + [pltpu.SemaphoreType.DMA for _ in range(1 + 2 * NRING)]
        ),
    )
    return run(bytes_, emb, pos).reshape(B, K, P * D_G)


def kernel(bytes, emb, pos):
    return _patch_embed(bytes, emb, pos)


# speedup vs baseline: 1.2342x; 1.0691x over previous
import jax
import jax.numpy as jnp
from jax import lax
from jax.experimental import pallas as pl
from jax.experimental.pallas import tpu as pltpu
from jax.experimental.pallas import tpu_sc as plsc

V = 256
D_G = 512
T = 2048
P = 16
K = 128
B = 4

SB = 2
TB = 512

_info = plsc.get_sparse_core_info()
NS, L = _info.num_subcores, _info.num_lanes
C = T // NS
R = 32
NSUB = (SB * C) // R
NRING = 3


def _sc_body(bytes_hbm, emb_hbm, pos_hbm, out_hbm,
             idx_buf, pbuf, ring0, ring1, ring2,
             psem, gsem0, gsem1, gsem2, osem0, osem1, osem2):
    s = lax.axis_index("s")
    t0 = s * C

    rings = (ring0, ring1, ring2)
    gsems = (gsem0, gsem1, gsem2)
    osems = (osem0, osem1, osem2)

    pos_dma = pltpu.async_copy(pos_hbm.at[pl.ds(t0, C)], pbuf, psem)
    for b in range(SB):
        pltpu.sync_copy(bytes_hbm.at[b, pl.ds(t0, C)], idx_buf.at[b])

    def issue_gather(sn):
        b, h = divmod(sn, C // R)
        slot = sn % NRING
        idx = idx_buf.at[b, pl.ds(h * R, R)]
        return pltpu.async_copy(emb_hbm.at[idx], rings[slot], gsems[slot])

    gather_dmas = [None] * NSUB
    out_dmas = [None] * NSUB
    for sn in range(2):
        gather_dmas[sn] = issue_gather(sn)
    pos_dma.wait()

    for sn in range(NSUB):
        b, h = divmod(sn, C // R)
        slot = sn % NRING
        buf = rings[slot]
        gather_dmas[sn].wait()

        @plsc.parallel_loop(0, R)
        def add_row(r, buf=buf, h=h):
            for j in range(D_G // L):
                sl = pl.ds(j * L, L)
                plsc.addupdate(buf.at[r, sl], pbuf[h * R + r, sl])

        out_dmas[sn] = pltpu.async_copy(
            buf, out_hbm.at[pl.ds(b * T + t0 + h * R, R)], osems[slot])

        nxt = sn + 2
        if nxt < NSUB:
            if nxt >= NRING:
                out_dmas[nxt - NRING].wait()
            gather_dmas[nxt] = issue_gather(nxt)

    for sn in range(NSUB - NRING, NSUB):
        if out_dmas[sn] is not None:
            out_dmas[sn].wait()


def _tc_body(dummy_ref, bytes_ref, emb_ref, pos_ref, out_ref):
    del dummy_ref
    jt = pl.program_id(0)
    bb = pl.program_id(1)
    ids = bytes_ref[bb, pl.ds(jt * TB, TB)]
    onehot = (ids[:, None] == lax.broadcasted_iota(jnp.int32, (TB, V), 1))
    gathered = jnp.dot(onehot.astype(jnp.bfloat16), emb_ref[...],
                       preferred_element_type=jnp.float32)
    out_ref[0] = gathered + pos_ref[...]


@jax.jit
def _patch_embed(bytes_, emb, pos):
    sc = pl.kernel(
        _sc_body,
        out_type=jax.ShapeDtypeStruct((B * T, D_G), jnp.float32),
        mesh=plsc.VectorSubcoreMesh(core_axis_name="c", subcore_axis_name="s",
                                    num_cores=1),
        scratch_types=(
            [pltpu.VMEM((SB, C), jnp.int32),
             pltpu.VMEM((C, D_G), jnp.float32)]
            + [pltpu.VMEM((R, D_G), jnp.float32) for _ in range(NRING)]
            + [pltpu.SemaphoreType.DMA for _ in range(1 + 2 * NRING)]
        ),
    )
    sc_full = sc(bytes_, emb, pos).reshape(B, T, D_G)

    out = pl.pallas_call(
        _tc_body,
        grid=(T // TB, B - SB),
        in_specs=[
            pl.BlockSpec((1, 8, 128), lambda jt, bb: (0, 0, 0)),
            pl.BlockSpec((B - SB, T), lambda jt, bb: (0, 0)),
            pl.BlockSpec((V, D_G), lambda jt, bb: (0, 0)),
            pl.BlockSpec((TB, D_G), lambda jt, bb: (jt, 0)),
        ],
        out_specs=pl.BlockSpec((1, TB, D_G), lambda jt, bb: (SB + bb, jt, 0)),
        out_shape=jax.ShapeDtypeStruct((B, T, D_G), jnp.float32),
        input_output_aliases={0: 0},
    )(sc_full, bytes_[SB:], emb.astype(jnp.bfloat16), pos)
    return out.reshape(B, K, P * D_G)


def kernel(bytes, emb, pos):
    return _patch_embed(bytes, emb, pos)


# speedup vs baseline: 1.3506x; 1.0944x over previous
import jax
import jax.numpy as jnp
from jax import lax
from jax.experimental import pallas as pl
from jax.experimental.pallas import tpu as pltpu
from jax.experimental.pallas import tpu_sc as plsc

V = 256
D_G = 512
T = 2048
P = 16
K = 128
B = 4

SB = 1
TB = 512

_info = plsc.get_sparse_core_info()
NS, L = _info.num_subcores, _info.num_lanes
C = T // NS
R = 32
NSUB = (SB * C) // R
NRING = 3


def _sc_body(bytes_hbm, emb_hbm, pos_hbm, out_hbm,
             idx_buf, pbuf, ring0, ring1, ring2,
             psem, gsem0, gsem1, gsem2, osem0, osem1, osem2):
    s = lax.axis_index("s")
    t0 = s * C

    rings = (ring0, ring1, ring2)
    gsems = (gsem0, gsem1, gsem2)
    osems = (osem0, osem1, osem2)

    pos_dma = pltpu.async_copy(pos_hbm.at[pl.ds(t0, C)], pbuf, psem)
    for b in range(SB):
        pltpu.sync_copy(bytes_hbm.at[b, pl.ds(t0, C)], idx_buf.at[b])

    def issue_gather(sn):
        b, h = divmod(sn, C // R)
        slot = sn % NRING
        idx = idx_buf.at[b, pl.ds(h * R, R)]
        return pltpu.async_copy(emb_hbm.at[idx], rings[slot], gsems[slot])

    gather_dmas = [None] * NSUB
    out_dmas = [None] * NSUB
    for sn in range(2):
        gather_dmas[sn] = issue_gather(sn)
    pos_dma.wait()

    for sn in range(NSUB):
        b, h = divmod(sn, C // R)
        slot = sn % NRING
        buf = rings[slot]
        gather_dmas[sn].wait()

        @plsc.parallel_loop(0, R)
        def add_row(r, buf=buf, h=h):
            for j in range(D_G // L):
                sl = pl.ds(j * L, L)
                plsc.addupdate(buf.at[r, sl], pbuf[h * R + r, sl])

        out_dmas[sn] = pltpu.async_copy(
            buf, out_hbm.at[pl.ds(b * T + t0 + h * R, R)], osems[slot])

        nxt = sn + 2
        if nxt < NSUB:
            if nxt >= NRING:
                out_dmas[nxt - NRING].wait()
            gather_dmas[nxt] = issue_gather(nxt)

    for sn in range(NSUB - NRING, NSUB):
        if out_dmas[sn] is not None:
            out_dmas[sn].wait()


def _tc_body(dummy_ref, bytes_ref, emb_ref, pos_ref, out_ref):
    del dummy_ref
    jt = pl.program_id(0)
    bb = pl.program_id(1)
    ids = bytes_ref[bb, pl.ds(jt * TB, TB)]
    onehot = (ids[:, None] == lax.broadcasted_iota(jnp.int32, (TB, V), 1))
    gathered = jnp.dot(onehot.astype(jnp.bfloat16), emb_ref[...],
                       preferred_element_type=jnp.float32)
    out_ref[0] = gathered + pos_ref[...]


@jax.jit
def _patch_embed(bytes_, emb, pos):
    sc = pl.kernel(
        _sc_body,
        out_type=jax.ShapeDtypeStruct((B * T, D_G), jnp.float32),
        mesh=plsc.VectorSubcoreMesh(core_axis_name="c", subcore_axis_name="s",
                                    num_cores=1),
        scratch_types=(
            [pltpu.VMEM((SB, C), jnp.int32),
             pltpu.VMEM((C, D_G), jnp.float32)]
            + [pltpu.VMEM((R, D_G), jnp.float32) for _ in range(NRING)]
            + [pltpu.SemaphoreType.DMA for _ in range(1 + 2 * NRING)]
        ),
    )
    sc_full = sc(bytes_, emb, pos).reshape(B, T, D_G)

    out = pl.pallas_call(
        _tc_body,
        grid=(T // TB, B - SB),
        in_specs=[
            pl.BlockSpec((1, 8, 128), lambda jt, bb: (0, 0, 0)),
            pl.BlockSpec((B - SB, T), lambda jt, bb: (0, 0)),
            pl.BlockSpec((V, D_G), lambda jt, bb: (0, 0)),
            pl.BlockSpec((TB, D_G), lambda jt, bb: (jt, 0)),
        ],
        out_specs=pl.BlockSpec((1, TB, D_G), lambda jt, bb: (SB + bb, jt, 0)),
        out_shape=jax.ShapeDtypeStruct((B, T, D_G), jnp.float32),
        input_output_aliases={0: 0},
    )(sc_full, bytes_[SB:], emb.astype(jnp.bfloat16), pos)
    return out.reshape(B, K, P * D_G)


def kernel(bytes, emb, pos):
    return _patch_embed(bytes, emb, pos)


# speedup vs baseline: 1.4018x; 1.0379x over previous
import jax
import jax.numpy as jnp
from jax import lax
from jax.experimental import pallas as pl
from jax.experimental.pallas import tpu as pltpu
from jax.experimental.pallas import tpu_sc as plsc

V = 256
D_G = 512
T = 2048
P = 16
K = 128
B = 4

SB = 1
TB = 1024

_info = plsc.get_sparse_core_info()
NS, L = _info.num_subcores, _info.num_lanes
C = T // NS
R = 32
NSUB = (SB * C) // R
NRING = 3


def _sc_body(bytes_hbm, emb_hbm, pos_hbm, out_hbm,
             idx_buf, pbuf, ring0, ring1, ring2,
             psem, gsem0, gsem1, gsem2, osem0, osem1, osem2):
    s = lax.axis_index("s")
    t0 = s * C

    rings = (ring0, ring1, ring2)
    gsems = (gsem0, gsem1, gsem2)
    osems = (osem0, osem1, osem2)

    pos_dma = pltpu.async_copy(pos_hbm.at[pl.ds(t0, C)], pbuf, psem)
    for b in range(SB):
        pltpu.sync_copy(bytes_hbm.at[b, pl.ds(t0, C)], idx_buf.at[b])

    def issue_gather(sn):
        b, h = divmod(sn, C // R)
        slot = sn % NRING
        idx = idx_buf.at[b, pl.ds(h * R, R)]
        return pltpu.async_copy(emb_hbm.at[idx], rings[slot], gsems[slot])

    gather_dmas = [None] * NSUB
    out_dmas = [None] * NSUB
    for sn in range(2):
        gather_dmas[sn] = issue_gather(sn)
    pos_dma.wait()

    for sn in range(NSUB):
        b, h = divmod(sn, C // R)
        slot = sn % NRING
        buf = rings[slot]
        gather_dmas[sn].wait()

        @plsc.parallel_loop(0, R)
        def add_row(r, buf=buf, h=h):
            for j in range(D_G // L):
                sl = pl.ds(j * L, L)
                plsc.addupdate(buf.at[r, sl], pbuf[h * R + r, sl])

        out_dmas[sn] = pltpu.async_copy(
            buf, out_hbm.at[pl.ds(b * T + t0 + h * R, R)], osems[slot])

        nxt = sn + 2
        if nxt < NSUB:
            if nxt >= NRING:
                out_dmas[nxt - NRING].wait()
            gather_dmas[nxt] = issue_gather(nxt)

    for sn in range(NSUB - NRING, NSUB):
        if out_dmas[sn] is not None:
            out_dmas[sn].wait()


def _tc_body(dummy_ref, bytes_ref, emb_ref, pos_ref, out_ref):
    del dummy_ref
    jt = pl.program_id(0)
    bb = pl.program_id(1)
    ids = bytes_ref[bb, pl.ds(jt * TB, TB)]
    onehot = (ids[:, None] == lax.broadcasted_iota(jnp.int32, (TB, V), 1))
    gathered = jnp.dot(onehot.astype(jnp.bfloat16), emb_ref[...],
                       preferred_element_type=jnp.float32)
    out_ref[0] = gathered + pos_ref[...]


@jax.jit
def _patch_embed(bytes_, emb, pos):
    sc = pl.kernel(
        _sc_body,
        out_type=jax.ShapeDtypeStruct((B * T, D_G), jnp.float32),
        mesh=plsc.VectorSubcoreMesh(core_axis_name="c", subcore_axis_name="s",
                                    num_cores=1),
        scratch_types=(
            [pltpu.VMEM((SB, C), jnp.int32),
             pltpu.VMEM((C, D_G), jnp.float32)]
            + [pltpu.VMEM((R, D_G), jnp.float32) for _ in range(NRING)]
            + [pltpu.SemaphoreType.DMA for _ in range(1 + 2 * NRING)]
        ),
    )
    sc_full = sc(bytes_, emb, pos).reshape(B, T, D_G)

    out = pl.pallas_call(
        _tc_body,
        grid=(T // TB, B - SB),
        in_specs=[
            pl.BlockSpec((1, 8, 128), lambda jt, bb: (0, 0, 0)),
            pl.BlockSpec((B - SB, T), lambda jt, bb: (0, 0)),
            pl.BlockSpec((V, D_G), lambda jt, bb: (0, 0)),
            pl.BlockSpec((TB, D_G), lambda jt, bb: (jt, 0)),
        ],
        out_specs=pl.BlockSpec((1, TB, D_G), lambda jt, bb: (SB + bb, jt, 0)),
        out_shape=jax.ShapeDtypeStruct((B, T, D_G), jnp.float32),
        input_output_aliases={0: 0},
    )(sc_full, bytes_[SB:], emb.astype(jnp.bfloat16), pos)
    return out.reshape(B, K, P * D_G)


def kernel(bytes, emb, pos):
    return _patch_embed(bytes, emb, pos)


# speedup vs baseline: 1.4948x; 1.0664x over previous
import jax
import jax.numpy as jnp
from jax import lax
from jax.experimental import pallas as pl
from jax.experimental.pallas import tpu as pltpu
from jax.experimental.pallas import tpu_sc as plsc

V = 256
D_G = 512
T = 2048
P = 16
K = 128
B = 4

SB = 1
TB = 1024

_info = plsc.get_sparse_core_info()
NS, L = _info.num_subcores, _info.num_lanes
C = T // NS
R = 32
NSUB = (SB * C) // R
NRING = 3


def _sc_body(bytes_hbm, emb_hbm, pos_hbm, out_hbm,
             idx_buf, pbuf, ring0, ring1, ring2,
             psem, gsem0, gsem1, gsem2, osem0, osem1, osem2):
    s = lax.axis_index("s")
    t0 = s * C

    rings = (ring0, ring1, ring2)
    gsems = (gsem0, gsem1, gsem2)
    osems = (osem0, osem1, osem2)

    pos_dma = pltpu.async_copy(pos_hbm.at[pl.ds(t0, C)], pbuf, psem)
    for b in range(SB):
        pltpu.sync_copy(bytes_hbm.at[b, pl.ds(t0, C)], idx_buf.at[b])

    def issue_gather(sn):
        b, h = divmod(sn, C // R)
        slot = sn % NRING
        idx = idx_buf.at[b, pl.ds(h * R, R)]
        return pltpu.async_copy(emb_hbm.at[idx], rings[slot], gsems[slot])

    gather_dmas = [None] * NSUB
    out_dmas = [None] * NSUB
    for sn in range(2):
        gather_dmas[sn] = issue_gather(sn)
    pos_dma.wait()

    for sn in range(NSUB):
        b, h = divmod(sn, C // R)
        slot = sn % NRING
        buf = rings[slot]
        gather_dmas[sn].wait()

        @plsc.parallel_loop(0, R)
        def add_row(r, buf=buf, h=h):
            for j in range(D_G // L):
                sl = pl.ds(j * L, L)
                plsc.addupdate(buf.at[r, sl], pbuf[h * R + r, sl])

        out_dmas[sn] = pltpu.async_copy(
            buf, out_hbm.at[pl.ds(b * T + t0 + h * R, R)], osems[slot])

        nxt = sn + 2
        if nxt < NSUB:
            if nxt >= NRING:
                out_dmas[nxt - NRING].wait()
            gather_dmas[nxt] = issue_gather(nxt)

    for sn in range(NSUB - NRING, NSUB):
        if out_dmas[sn] is not None:
            out_dmas[sn].wait()


def _tc_body(bytes_ref, emb_ref, pos_ref, out_ref):
    jt = pl.program_id(0)
    bb = pl.program_id(1)
    ids = bytes_ref[bb, pl.ds(jt * TB, TB)]
    onehot = (ids[:, None] == lax.broadcasted_iota(jnp.int32, (TB, V), 1))
    gathered = jnp.dot(onehot.astype(jnp.bfloat16), emb_ref[...],
                       preferred_element_type=jnp.float32)
    out_ref[0] = gathered + pos_ref[...]


def _inj_body(dummy_ref, sc_ref, out_ref):
    del dummy_ref
    out_ref[0] = sc_ref[...]


@jax.jit
def _patch_embed(bytes_, emb, pos):
    sc = pl.kernel(
        _sc_body,
        out_type=jax.ShapeDtypeStruct((SB * T, D_G), jnp.float32),
        mesh=plsc.VectorSubcoreMesh(core_axis_name="c", subcore_axis_name="s",
                                    num_cores=1),
        scratch_types=(
            [pltpu.VMEM((SB, C), jnp.int32),
             pltpu.VMEM((C, D_G), jnp.float32)]
            + [pltpu.VMEM((R, D_G), jnp.float32) for _ in range(NRING)]
            + [pltpu.SemaphoreType.DMA for _ in range(1 + 2 * NRING)]
        ),
    )
    sc_out = sc(bytes_, emb, pos)

    tc_full = pl.pallas_call(
        _tc_body,
        grid=(T // TB, B - SB),
        in_specs=[
            pl.BlockSpec((B - SB, T), lambda jt, bb: (0, 0)),
            pl.BlockSpec((V, D_G), lambda jt, bb: (0, 0)),
            pl.BlockSpec((TB, D_G), lambda jt, bb: (jt, 0)),
        ],
        out_specs=pl.BlockSpec((1, TB, D_G), lambda jt, bb: (SB + bb, jt, 0)),
        out_shape=jax.ShapeDtypeStruct((B, T, D_G), jnp.float32),
    )(bytes_[SB:], emb.astype(jnp.bfloat16), pos)

    out = pl.pallas_call(
        _inj_body,
        grid=(SB * T // TB,),
        in_specs=[
            pl.BlockSpec((1, 8, 128), lambda jt: (0, 0, 0)),
            pl.BlockSpec((TB, D_G), lambda jt: (jt, 0)),
        ],
        out_specs=pl.BlockSpec((1, TB, D_G), lambda jt: (0, jt, 0)),
        out_shape=jax.ShapeDtypeStruct((B, T, D_G), jnp.float32),
        input_output_aliases={0: 0},
    )(tc_full, sc_out)
    return out.reshape(B, K, P * D_G)


def kernel(bytes, emb, pos):
    return _patch_embed(bytes, emb, pos)
